# Initial kernel scaffold; baseline (speedup 1.0000x reference)
import jax
import jax.numpy as jnp
from jax.experimental import pallas as pl


def kernel(node_features, edge_index, emb_W, emb_b, emb_g, emb_be, emb_rm, emb_rv, msg_W, msg_b, att_W, att_b, gW_ih, gb_ih, gW_hh, gb_hh, bn_g, bn_be, bn_rm, bn_rv, ro_W1, ro_b1, ro_W2, ro_b2):
    raise NotImplementedError("write your pallas kernel here")


# speedup vs baseline: 5.6020x; 5.6020x over previous
import functools

import jax
import jax.numpy as jnp
from jax import lax
from jax.experimental import pallas as pl
from jax.experimental.pallas import tpu as pltpu
from jax.experimental.pallas import tpu_sc as plsc

N = 10000
E = 320000
H = 128
L = 4
EPS = 1e-5

BN = 2000
GRID = N // BN

NC = 2
NS = 16
NW = NC * NS
EPW = E // NW
K = 80
NCH = EPW // K
NPAD = 10240
RPT = NPAD // NS


def _bn_apply(h, g, be, rm, rv):
    return (h - rm) * (g * lax.rsqrt(rv + EPS)) + be


def _sigmoid(t):
    return 1.0 / (1.0 + jnp.exp(-t))


def _embed_pre_body(nf, embWt, embb, embg, embbe, embrm, embrv,
                    msgWt, msgb, attw, attb2, x_o, m_o, a2_o):
    x = jnp.maximum(jnp.dot(nf[...], embWt[...]) + embb[...], 0.0)
    x = _bn_apply(x, embg[...], embbe[...], embrm[...], embrv[...])
    x_o[...] = x
    m_o[...] = jnp.dot(x, msgWt[...]) + msgb[...]
    a2_o[...] = jnp.dot(x, attw[...]) + attb2[...]


def _gru_core(agg2, x, gWiht, gbih, gWhht, gbhh, bng, bnbe, bnrm, bnrv):
    agg = agg2[0] + agg2[1]
    gi = jnp.dot(agg, gWiht) + gbih
    gh = jnp.dot(x, gWhht) + gbhh
    r = _sigmoid(gi[:, :H] + gh[:, :H])
    z = _sigmoid(gi[:, H:2 * H] + gh[:, H:2 * H])
    n = jnp.tanh(gi[:, 2 * H:] + r * gh[:, 2 * H:])
    h = (1.0 - z) * n + z * x
    return _bn_apply(h, bng, bnbe, bnrm, bnrv)


def _gru_pre_body(agg2, x, gWiht, gbih, gWhht, gbhh, bng, bnbe, bnrm, bnrv,
                  msgWt, msgb, attw, attb2, x_o, m_o, a2_o):
    xn = _gru_core(agg2[...], x[...], gWiht[...], gbih[...], gWhht[...],
                   gbhh[...], bng[...], bnbe[...], bnrm[...], bnrv[...])
    x_o[...] = xn
    m_o[...] = jnp.dot(xn, msgWt[...]) + msgb[...]
    a2_o[...] = jnp.dot(xn, attw[...]) + attb2[...]


def _gru_final_body(agg2, x, gWiht, gbih, gWhht, gbhh, bng, bnbe, bnrm, bnrv,
                    roW1t, rob1, roW2t, rob2, out_o, gsum):
    i = pl.program_id(0)
    xn = _gru_core(agg2[...], x[...], gWiht[...], gbih[...], gWhht[...],
                   gbhh[...], bng[...], bnbe[...], bnrm[...], bnrv[...])

    @pl.when(i == 0)
    def _():
        gsum[...] = jnp.zeros_like(gsum)

    gsum[...] += jnp.sum(xn, axis=0, keepdims=True)
    h1 = jnp.maximum(jnp.dot(gsum[...], roW1t[...]) + rob1[...], 0.0)
    out_o[...] = jnp.dot(h1, roW2t[...]) + rob2[...]


def _row_spec(w=H):
    return pl.BlockSpec((BN, w), lambda i: (i, 0))


def _full_spec(shape):
    nd = len(shape)
    return pl.BlockSpec(shape, lambda i: (0,) * nd)


def _embed_pre_call(nf, embWt, embb, embg, embbe, embrm, embrv,
                    msgWt, msgb, attw, attb2):
    return pl.pallas_call(
        _embed_pre_body,
        grid=(GRID,),
        in_specs=[
            _row_spec(), _full_spec((H, H)), _full_spec((1, H)),
            _full_spec((1, H)), _full_spec((1, H)), _full_spec((1, H)),
            _full_spec((1, H)), _full_spec((H, H)), _full_spec((1, H)),
            _full_spec((H, 2)), _full_spec((1, 2)),
        ],
        out_specs=[_row_spec(), _row_spec(), _row_spec(2)],
        out_shape=[
            jax.ShapeDtypeStruct((N, H), jnp.float32),
            jax.ShapeDtypeStruct((N, H), jnp.float32),
            jax.ShapeDtypeStruct((N, 2), jnp.float32),
        ],
    )(nf, embWt, embb, embg, embbe, embrm, embrv, msgWt, msgb, attw, attb2)


def _gru_pre_call(agg2, x, gWiht, gbih, gWhht, gbhh, bng, bnbe, bnrm, bnrv,
                  msgWt, msgb, attw, attb2):
    return pl.pallas_call(
        _gru_pre_body,
        grid=(GRID,),
        in_specs=[
            pl.BlockSpec((2, BN, H), lambda i: (0, i, 0)), _row_spec(),
            _full_spec((H, 3 * H)), _full_spec((1, 3 * H)),
            _full_spec((H, 3 * H)), _full_spec((1, 3 * H)),
            _full_spec((1, H)), _full_spec((1, H)), _full_spec((1, H)),
            _full_spec((1, H)), _full_spec((H, H)), _full_spec((1, H)),
            _full_spec((H, 2)), _full_spec((1, 2)),
        ],
        out_specs=[_row_spec(), _row_spec(), _row_spec(2)],
        out_shape=[
            jax.ShapeDtypeStruct((N, H), jnp.float32),
            jax.ShapeDtypeStruct((N, H), jnp.float32),
            jax.ShapeDtypeStruct((N, 2), jnp.float32),
        ],
    )(agg2, x, gWiht, gbih, gWhht, gbhh, bng, bnbe, bnrm, bnrv,
      msgWt, msgb, attw, attb2)


def _gru_final_call(agg2, x, gWiht, gbih, gWhht, gbhh, bng, bnbe, bnrm, bnrv,
                    roW1t, rob1, roW2t, rob2):
    return pl.pallas_call(
        _gru_final_body,
        grid=(GRID,),
        in_specs=[
            pl.BlockSpec((2, BN, H), lambda i: (0, i, 0)), _row_spec(),
            _full_spec((H, 3 * H)), _full_spec((1, 3 * H)),
            _full_spec((H, 3 * H)), _full_spec((1, 3 * H)),
            _full_spec((1, H)), _full_spec((1, H)), _full_spec((1, H)),
            _full_spec((1, H)), _full_spec((H, H // 2)),
            _full_spec((1, H // 2)), _full_spec((H // 2, 1)),
            _full_spec((1, 1)),
        ],
        out_specs=_full_spec((1, 1)),
        out_shape=jax.ShapeDtypeStruct((1, 1), jnp.float32),
        scratch_shapes=[pltpu.VMEM((1, H), jnp.float32)],
    )(agg2, x, gWiht, gbih, gWhht, gbhh, bng, bnbe, bnrm, bnrv,
      roW1t, rob1, roW2t, rob2)


def _sc_agg_body(m_hbm, src_hbm, dst_hbm, as_hbm, ad_hbm, out_hbm,
                 agg_sh, src_v, dst_v, rows_v, gs_v, gd_v, bnc_v, sem):
    c = lax.axis_index("c")
    s = lax.axis_index("s")
    g = c * NS + s

    def zrow(i, carry):
        for j in range(8):
            bnc_v[i, pl.ds(j * 16, 16)] = jnp.zeros((16,), jnp.float32)
        return carry

    lax.fori_loop(0, 64, zrow, 0)
    for j in range(10):
        pltpu.sync_copy(bnc_v, agg_sh.at[pl.ds(s * RPT + j * 64, 64)])
    plsc.subcore_barrier()

    def chunk(ci, carry):
        base = g * EPW + ci * K
        pltpu.sync_copy(src_hbm.at[pl.ds(base, K)], src_v)
        pltpu.sync_copy(dst_hbm.at[pl.ds(base, K)], dst_v)
        cp0 = pltpu.async_copy(m_hbm.at[src_v], rows_v, sem)
        cp1 = pltpu.async_copy(as_hbm.at[src_v], gs_v, sem)
        cp2 = pltpu.async_copy(ad_hbm.at[dst_v], gd_v, sem)
        cp0.wait()
        cp1.wait()
        cp2.wait()
        for t in range(K // 16):
            w16 = _sigmoid(gs_v[pl.ds(t * 16, 16)] + gd_v[pl.ds(t * 16, 16)])
            for i in range(16):
                r = t * 16 + i
                wi = w16[i]
                for j in range(8):
                    rows_v[r, pl.ds(j * 16, 16)] = (
                        rows_v[r, pl.ds(j * 16, 16)] * wi)
        pltpu.sync_copy(rows_v, agg_sh.at[dst_v], add=True)
        return carry

    lax.fori_loop(0, NCH, chunk, 0)
    plsc.subcore_barrier()

    for j in range(10):
        pltpu.sync_copy(agg_sh.at[pl.ds(s * RPT + j * 64, 64)], bnc_v)
        pltpu.sync_copy(bnc_v, out_hbm.at[c, pl.ds(s * RPT + j * 64, 64)])


def _make_sc_agg():
    mesh = plsc.VectorSubcoreMesh(core_axis_name="c", subcore_axis_name="s",
                                  num_cores=NC, num_subcores=NS)
    return pl.kernel(
        _sc_agg_body,
        out_type=jax.ShapeDtypeStruct((NC, NPAD, H), jnp.float32),
        mesh=mesh,
        scratch_types=[
            pltpu.VMEM_SHARED((NPAD, H), jnp.float32),
            pltpu.VMEM((K,), jnp.int32),
            pltpu.VMEM((K,), jnp.int32),
            pltpu.VMEM((K, H), jnp.float32),
            pltpu.VMEM((K,), jnp.float32),
            pltpu.VMEM((K,), jnp.float32),
            pltpu.VMEM((64, H), jnp.float32),
            pltpu.SemaphoreType.DMA,
        ],
    )


def kernel(node_features, edge_index, emb_W, emb_b, emb_g, emb_be, emb_rm,
           emb_rv, msg_W, msg_b, att_W, att_b, gW_ih, gb_ih, gW_hh, gb_hh,
           bn_g, bn_be, bn_rm, bn_rv, ro_W1, ro_b1, ro_W2, ro_b2):
    ei = edge_index.astype(jnp.int32)
    src = ei[0]
    dst = ei[1]

    embWt = emb_W.T
    msgWt = [msg_W[l].T for l in range(L)]
    attw = [jnp.stack([att_W[l, 0, :H], att_W[l, 0, H:]], axis=1)
            for l in range(L)]
    zero11 = jnp.zeros((1, 1), jnp.float32)
    attb2 = [jnp.concatenate([zero11, att_b[l].reshape(1, 1)], axis=1)
             for l in range(L)]
    gWiht = [gW_ih[l].T for l in range(L)]
    gWhht = [gW_hh[l].T for l in range(L)]

    def r1(v):
        return v.reshape(1, -1)

    sc_agg = _make_sc_agg()

    x, m, a2 = _embed_pre_call(
        node_features, embWt, r1(emb_b), r1(emb_g), r1(emb_be), r1(emb_rm),
        r1(emb_rv), msgWt[0], r1(msg_b[0]), attw[0], attb2[0])

    out = None
    for l in range(L):
        a_s = a2[:, 0]
        a_d = a2[:, 1]
        agg2 = sc_agg(m, src, dst, a_s, a_d)
        if l < L - 1:
            x, m, a2 = _gru_pre_call(
                agg2, x, gWiht[l], r1(gb_ih[l]), gWhht[l], r1(gb_hh[l]),
                r1(bn_g[l]), r1(bn_be[l]), r1(bn_rm[l]), r1(bn_rv[l]),
                msgWt[l + 1], r1(msg_b[l + 1]), attw[l + 1], attb2[l + 1])
        else:
            out = _gru_final_call(
                agg2, x, gWiht[l], r1(gb_ih[l]), gWhht[l], r1(gb_hh[l]),
                r1(bn_g[l]), r1(bn_be[l]), r1(bn_rm[l]), r1(bn_rv[l]),
                ro_W1.T, r1(ro_b1), ro_W2.T, ro_b2.reshape(1, 1))
    return out
